# Initial kernel scaffold; baseline (speedup 1.0000x reference)
import jax
import jax.numpy as jnp
from jax.experimental import pallas as pl


def kernel(logits, targets, data_parameter_minibatch, exp_avg, index_dataset, epoch):
    raise NotImplementedError("write your pallas kernel here")


# speedup vs baseline: 2.9388x; 2.9388x over previous
import functools

import jax
import jax.numpy as jnp
from jax import lax
from jax.experimental import pallas as pl
from jax.experimental.pallas import tpu as pltpu
from jax.experimental.pallas import tpu_sc as plsc

B = 16384
C = 1000
M = 1000000
NW = 32
BPW = B // NW
LANES = 16
GROUPS = B // LANES
CH = 31256
LAST = M - (NW - 1) * CH
BETA = 0.9

_COLS = 2048


def _loss_body(xt_ref, t_ref, o_ref):
    x = xt_ref[...]
    t = t_ref[...].reshape(1, _COLS)
    m = jnp.max(x, axis=0, keepdims=True)
    lse = m + jnp.log(jnp.sum(jnp.exp(x - m), axis=0, keepdims=True))
    row = lax.broadcasted_iota(jnp.int32, x.shape, 0)
    tv = jnp.sum(jnp.where(row == t, x, 0.0), axis=0, keepdims=True)
    o_ref[...] = (lse - tv).reshape(_COLS // 128, 128)


def _ce_loss(logits, targets):
    return pl.pallas_call(
        _loss_body,
        grid=(B // _COLS,),
        in_specs=[
            pl.BlockSpec((C, _COLS), lambda i: (0, i)),
            pl.BlockSpec((_COLS // 128, 128), lambda i: (i, 0)),
        ],
        out_specs=pl.BlockSpec((_COLS // 128, 128), lambda i: (i, 0)),
        out_shape=jax.ShapeDtypeStruct((B // 128, 128), jnp.float32),
    )(logits.T, targets.reshape(B // 128, 128))


_IDX_ROWS_PER_W = (B // 128) // NW


def _sc_gather_body(table_hbm, idx_hbm, out_hbm, idx_v, g_v, sem):
    wid = lax.axis_index("s") * 2 + lax.axis_index("c")
    r0 = wid * _IDX_ROWS_PER_W
    pltpu.sync_copy(idx_hbm.at[pl.ds(r0, _IDX_ROWS_PER_W)], idx_v)
    descs = [
        pltpu.async_copy(table_hbm.at[idx_v.at[j]], g_v.at[j], sem)
        for j in range(_IDX_ROWS_PER_W)
    ]
    for d in descs:
        d.wait()
    pltpu.sync_copy(g_v, out_hbm.at[pl.ds(r0, _IDX_ROWS_PER_W)])


HALF = M // 2
STRIPE = 31248
HSTRIPE = STRIPE // 2
TAILW = HALF - 16 * STRIPE
SLICE = B // 16
SROWS = SLICE // 128
TRASH = HALF
ACC0 = HALF + 8


def _sc_update_body(exp_hbm, pk_hbm, dpm_hbm, params_hbm,
                    nl_out_hbm, exp_out_hbm,
                    pk_v, nl_v, sidx_v, dpm_v, out_v,
                    acc_v, aidx_v, accrd_v, params_v, stripe_v, tail_v,
                    img_sh, sem):
    c = lax.axis_index("c")
    s = lax.axis_index("s")
    half_lo = c * HALF
    r0 = s * SROWS

    stripe_lo = s * STRIPE
    ld = pltpu.async_copy(exp_hbm.at[pl.ds(half_lo + stripe_lo, STRIPE)],
                          stripe_v, sem)

    pltpu.sync_copy(pk_hbm.at[:, pl.ds(r0, SROWS)], pk_v)
    pltpu.sync_copy(params_hbm, params_v)
    pltpu.sync_copy(dpm_hbm.at[pl.ds(r0 + c * (SROWS // 2), SROWS // 2)],
                    dpm_v)

    lane = lax.iota(jnp.int32, LANES)
    p = params_v[...]
    inv_bc = jnp.sum(jnp.where(lane == 0, p, 0.0))
    es = jnp.sum(jnp.where(lane == 1, p, 0.0))

    acc = jnp.zeros((LANES,), jnp.float32)
    for r in range(SROWS):
        def grp(j, a, r=r):
            sl = pl.ds(j * LANES, LANES)
            fsl = pl.ds(r * 128 + j * LANES, LANES)
            ii = plsc.bitcast(pk_v[0, r, sl], jnp.int32)
            nl = pk_v[1, r, sl] * BETA + pk_v[2, r, sl] * (1.0 - BETA)
            nl_v[fsl] = nl
            li = ii - half_lo
            m = (li >= 0) & (li < HALF)
            sidx_v[fsl] = jnp.where(m, li, TRASH)
            return a + nl

        acc = lax.fori_loop(0, 128 // LANES, grp, acc)

    lane16 = lax.iota(jnp.int32, LANES)
    aidx_v[...] = lane16 + ACC0

    @pl.when(s == 0)
    def _():
        accrd_v[...] = jnp.zeros((LANES,), jnp.float32)
        pltpu.sync_copy(accrd_v, img_sh.at[pl.ds(ACC0, LANES)])

    ld.wait()
    pltpu.sync_copy(stripe_v, img_sh.at[pl.ds(stripe_lo, STRIPE)])

    @pl.when(s == 15)
    def _():
        pltpu.sync_copy(exp_hbm.at[pl.ds(half_lo + 16 * STRIPE, TAILW)],
                        tail_v)
        pltpu.sync_copy(tail_v, img_sh.at[pl.ds(16 * STRIPE, TAILW)])

    plsc.subcore_barrier()

    acc_v[...] = acc
    pltpu.sync_copy(acc_v, img_sh.at[aidx_v], add=True)

    pltpu.sync_copy(nl_v, img_sh.at[sidx_v])

    plsc.subcore_barrier()

    pltpu.sync_copy(img_sh.at[pl.ds(stripe_lo, STRIPE)], stripe_v)
    st = pltpu.async_copy(stripe_v,
                          exp_out_hbm.at[pl.ds(half_lo + stripe_lo, STRIPE)],
                          sem)

    @pl.when(s == 15)
    def _():
        pltpu.sync_copy(img_sh.at[pl.ds(16 * STRIPE, TAILW)], tail_v)
        pltpu.sync_copy(tail_v,
                        exp_out_hbm.at[pl.ds(half_lo + 16 * STRIPE, TAILW)])

    pltpu.sync_copy(img_sh.at[pl.ds(ACC0, LANES)], accrd_v)
    k1 = jnp.sum(accrd_v[...]) * inv_bc * (1.0 / B)

    for cc in range(2):
        @pl.when(c == cc)
        def _(cc=cc):
            for r in range(SROWS // 2):
                def grp2(j, carry, r=r, cc=cc):
                    sl = pl.ds(j * LANES, LANES)
                    nl = nl_v[pl.ds((cc * (SROWS // 2) + r) * 128 + j * LANES,
                                    LANES)]
                    out_v[r, sl] = (nl * inv_bc - k1) * es / dpm_v[r, sl]
                    return carry

                lax.fori_loop(0, 128 // LANES, grp2, 0)

    pltpu.sync_copy(out_v, nl_out_hbm.at[pl.ds(r0 + c * (SROWS // 2),
                                               SROWS // 2)])
    st.wait()


@functools.lru_cache(maxsize=1)
def _sc_kernels():
    mesh = plsc.VectorSubcoreMesh(core_axis_name="c", subcore_axis_name="s")
    params = pltpu.CompilerParams(needs_layout_passes=False)
    gather = pl.kernel(
        _sc_gather_body,
        out_type=jax.ShapeDtypeStruct((B // 128, 128), jnp.float32),
        mesh=mesh,
        scratch_types=[
            pltpu.VMEM((_IDX_ROWS_PER_W, 128), jnp.int32),
            pltpu.VMEM((_IDX_ROWS_PER_W, 128), jnp.float32),
            pltpu.SemaphoreType.DMA,
        ],
        compiler_params=params,
    )
    update = pl.kernel(
        _sc_update_body,
        out_type=(
            jax.ShapeDtypeStruct((B // 128, 128), jnp.float32),
            jax.ShapeDtypeStruct((M,), jnp.float32),
        ),
        mesh=mesh,
        scratch_types=[
            pltpu.VMEM((3, SROWS, 128), jnp.float32),
            pltpu.VMEM((SLICE,), jnp.float32),
            pltpu.VMEM((SLICE,), jnp.int32),
            pltpu.VMEM((SROWS // 2, 128), jnp.float32),
            pltpu.VMEM((SROWS // 2, 128), jnp.float32),
            pltpu.VMEM((LANES,), jnp.float32),
            pltpu.VMEM((LANES,), jnp.int32),
            pltpu.VMEM((LANES,), jnp.float32),
            pltpu.VMEM((LANES,), jnp.float32),
            pltpu.VMEM((STRIPE,), jnp.float32),
            pltpu.VMEM((TAILW,), jnp.float32),
            pltpu.VMEM_SHARED((HALF + 32,), jnp.float32),
            pltpu.SemaphoreType.DMA,
        ],
        compiler_params=params,
    )
    return gather, update


def kernel(logits, targets, data_parameter_minibatch, exp_avg, index_dataset,
           epoch):
    epoch = jnp.asarray(epoch)
    ef = (epoch + 1).astype(jnp.float32)
    bias_cor = 1.0 - jnp.power(jnp.float32(BETA), ef)
    es = jnp.where(epoch < 10, ef / 10.0, 1.0).astype(jnp.float32)
    params = jnp.zeros((LANES,), jnp.float32)
    params = params.at[0].set(1.0 / bias_cor).at[1].set(es)

    sc_gather, sc_update = _sc_kernels()
    idx2d = index_dataset.reshape(B // 128, 128)
    loss2d = _ce_loss(logits, targets)
    g2d = sc_gather(exp_avg, idx2d)
    idxf = lax.bitcast_convert_type(idx2d, jnp.float32)
    packed = jnp.stack([idxf, g2d, loss2d], axis=0)
    new_loss2d, exp_avg_new = sc_update(
        exp_avg, packed,
        data_parameter_minibatch.reshape(B // 128, 128), params)
    return new_loss2d.reshape(B), exp_avg_new
